# baseline (speedup 1.0000x reference)
import functools

import jax
import jax.numpy as jnp
from jax import lax
from jax.experimental import pallas as pl
from jax.experimental.pallas import tpu as pltpu
from jax.experimental.pallas import tpu_sc as plsc

N = 10000
E = 320000
ALPHA = 0.2

NC = 2
NS = 16
NW = NC * NS
EPT = 10240
CH = 128
NCHUNK = EPT // CH
NPAD = 10240
TRASH = N
ZROWS = NPAD // NS
DRAIN_A = 624
DRAIN_LAST = N - 15 * DRAIN_A


def _drain(acc, out2d, s):
    @pl.when(s < NS - 1)
    def _():
        pltpu.sync_copy(acc.at[pl.ds(s * DRAIN_A, DRAIN_A)],
                        out2d.at[pl.ds(s * DRAIN_A, DRAIN_A)])

    @pl.when(s == NS - 1)
    def _():
        pltpu.sync_copy(acc.at[pl.ds(15 * DRAIN_A, DRAIN_LAST)],
                        out2d.at[pl.ds(15 * DRAIN_A, DRAIN_LAST)])

@functools.cache
def _mesh():
    return plsc.VectorSubcoreMesh(core_axis_name="c", subcore_axis_name="s",
                                  num_cores=NC, num_subcores=NS)


@functools.cache
def _make_sc_app(feature_split):
    w = 64
    ept = EPT * NC if feature_split else EPT
    nchunk = ept // CH
    BLK = 20
    nblk = nchunk // BLK
    R = 4

    @functools.partial(
        pl.kernel,
        out_type=jax.ShapeDtypeStruct((NC, N, w), jnp.float32),
        mesh=_mesh(),
        compiler_params=pltpu.CompilerParams(use_tc_tiling_on_sc=False),
        scratch_types=[
            pltpu.VMEM_SHARED((NPAD, w), jnp.float32),
            pltpu.VMEM_SHARED((N, w), jnp.float32),
            pltpu.VMEM((BLK, CH), jnp.int32),
            pltpu.VMEM((BLK, CH), jnp.int32),
            pltpu.VMEM((BLK, CH), jnp.int32),
            pltpu.VMEM((BLK, CH), jnp.int32),
            [pltpu.VMEM((CH, w), jnp.float32)] * R,
            [pltpu.SemaphoreType.DMA] * R,
            [pltpu.SemaphoreType.DMA] * R,
            pltpu.SemaphoreType.DMA,
            pltpu.SemaphoreType.DMA,
            pltpu.SemaphoreType.DMA,
        ],
    )
    def app(t_hbm, src_hbm, dst_hbm, p_hbm, acc, stage, sA, dA, sB, dB,
            bufs, gs, ss, isA, isB, sts):
        c = lax.axis_index("c")
        s = lax.axis_index("s")
        src_rows = src_hbm.at[c].at[s]
        dst_rows = dst_hbm.at[c].at[s]
        tsrc = t_hbm.at[c] if feature_split else t_hbm
        buf0 = bufs[0]

        @pl.when(s < NS - 1)
        def _():
            pltpu.async_copy(tsrc.at[pl.ds(s * DRAIN_A, DRAIN_A)],
                             stage.at[pl.ds(s * DRAIN_A, DRAIN_A)], sts)

        @pl.when(s == NS - 1)
        def _():
            pltpu.async_copy(tsrc.at[pl.ds(15 * DRAIN_A, DRAIN_LAST)],
                             stage.at[pl.ds(15 * DRAIN_A, DRAIN_LAST)], sts)

        @pl.loop(0, CH)
        def _(r):
            for j in range(w // 16):
                buf0[r, pl.ds(j * 16, 16)] = jnp.zeros((16,), jnp.float32)

        @pl.loop(0, ZROWS // CH)
        def _(z):
            pltpu.sync_copy(buf0, acc.at[pl.ds(s * ZROWS + z * CH, CH)])

        @pl.when(s < NS - 1)
        def _():
            pltpu.make_async_copy(tsrc.at[pl.ds(s * DRAIN_A, DRAIN_A)],
                                  stage.at[pl.ds(s * DRAIN_A, DRAIN_A)],
                                  sts).wait()

        @pl.when(s == NS - 1)
        def _():
            pltpu.make_async_copy(tsrc.at[pl.ds(15 * DRAIN_A, DRAIN_LAST)],
                                  stage.at[pl.ds(15 * DRAIN_A, DRAIN_LAST)],
                                  sts).wait()

        plsc.subcore_barrier()

        def load_idx(blk, sref, dref, isem):
            pltpu.async_copy(src_rows.at[pl.ds(blk * BLK, BLK)], sref, isem)
            pltpu.async_copy(dst_rows.at[pl.ds(blk * BLK, BLK)], dref, isem)

        def wait_idx(sref, dref, isem):
            pltpu.make_async_copy(src_rows.at[pl.ds(0, BLK)], sref,
                                  isem).wait()
            pltpu.make_async_copy(dst_rows.at[pl.ds(0, BLK)], dref,
                                  isem).wait()

        def process_block(sref, dref):
            for j in range(R):
                pltpu.async_copy(stage.at[sref.at[j]], bufs[j], gs[j])

            @pl.loop(0, BLK - R, step=R)
            def _(ch):
                for j in range(R):
                    pltpu.make_async_copy(stage.at[sref.at[j]], bufs[j],
                                          gs[j]).wait()
                    pltpu.async_copy(bufs[j], acc.at[dref.at[ch + j]],
                                     ss[j], add=True)
                for j in range(R):
                    pltpu.make_async_copy(bufs[j], acc.at[dref.at[ch + j]],
                                          ss[j]).wait()
                    pltpu.async_copy(stage.at[sref.at[ch + R + j]], bufs[j],
                                     gs[j])

            for j in range(R):
                pltpu.make_async_copy(stage.at[sref.at[j]], bufs[j],
                                      gs[j]).wait()
                pltpu.async_copy(bufs[j], acc.at[dref.at[BLK - R + j]],
                                 ss[j], add=True)
            for j in range(R):
                pltpu.make_async_copy(bufs[j], acc.at[dref.at[j]],
                                      ss[j]).wait()

        load_idx(0, sA, dA, isA)
        load_idx(1, sB, dB, isB)

        @pl.loop(0, nblk, step=2)
        def _(b):
            wait_idx(sA, dA, isA)
            process_block(sA, dA)

            @pl.when(b + 2 < nblk)
            def _():
                load_idx(b + 2, sA, dA, isA)

            wait_idx(sB, dB, isB)
            process_block(sB, dB)

            @pl.when(b + 3 < nblk)
            def _():
                load_idx(b + 3, sB, dB, isB)

        plsc.subcore_barrier()
        _drain(acc, p_hbm.at[c], s)

    return app


@functools.cache
def _make_sc_deg():
    @functools.partial(
        pl.kernel,
        out_type=jax.ShapeDtypeStruct((NC, N, 16), jnp.float32),
        mesh=_mesh(),
        compiler_params=pltpu.CompilerParams(use_tc_tiling_on_sc=False),
        scratch_types=[
            pltpu.VMEM_SHARED((NPAD, 16), jnp.float32),
            pltpu.VMEM((NCHUNK, CH), jnp.int32),
            pltpu.VMEM((CH, 16), jnp.float32),
            pltpu.SemaphoreType.DMA,
        ],
    )
    def _sc_deg(dst_hbm, out_hbm, acc, dst_v, ones_v, ssem):
        c = lax.axis_index("c")
        s = lax.axis_index("s")

        @pl.loop(0, CH)
        def _(r):
            ones_v[r, pl.ds(0, 16)] = jnp.zeros((16,), jnp.float32)

        @pl.loop(0, ZROWS // CH)
        def _(z):
            pltpu.sync_copy(ones_v, acc.at[pl.ds(s * ZROWS + z * CH, CH)])

        @pl.loop(0, CH)
        def _(r):
            ones_v[r, pl.ds(0, 16)] = jnp.ones((16,), jnp.float32)

        pltpu.sync_copy(dst_hbm.at[c].at[s], dst_v)
        plsc.subcore_barrier()

        @pl.loop(0, NCHUNK, step=8)
        def _(ch):
            for j in range(8):
                pltpu.async_copy(ones_v, acc.at[dst_v.at[ch + j]], ssem,
                                 add=True)
            for j in range(8):
                pltpu.make_async_copy(ones_v, acc.at[dst_v.at[ch + j]],
                                      ssem).wait()

        plsc.subcore_barrier()
        _drain(acc, out_hbm.at[c], s)

    return _sc_deg


B = 2000
GRID = (N // B,)


def _dinv(degp):
    deg = degp[0, :, 0] + degp[1, :, 0]
    return jnp.where(deg > 0, 1.0 / jnp.sqrt(jnp.maximum(deg, 1.0)),
                     0.0)[:, None]


def _leaky(x, a):
    return jnp.where(x >= 0, x, a * x)


def _split64(o_ref, v):
    o_ref[0] = v[:, 0:64]
    o_ref[1] = v[:, 64:128]


def _scale0_body(x_ref, degp_ref, o_ref):
    _split64(o_ref, _dinv(degp_ref[...]) * x_ref[...])


_scale0 = pl.pallas_call(
    _scale0_body,
    grid=GRID,
    in_specs=[
        pl.BlockSpec((B, 128), lambda i: (i, 0)),
        pl.BlockSpec((2, B, 16), lambda i: (0, i, 0)),
    ],
    out_specs=pl.BlockSpec((2, B, 64), lambda i: (0, i, 0)),
    out_shape=jax.ShapeDtypeStruct((NC, N, 64), jnp.float32),
)


def _comb1_body(p_ref, degp_ref, t_ref, u_ref):
    di = _dinv(degp_ref[...])
    t1 = -(di * jnp.concatenate([p_ref[0], p_ref[1]], axis=1))
    t_ref[...] = t1
    _split64(u_ref, di * t1)


_comb1 = pl.pallas_call(
    _comb1_body,
    grid=GRID,
    in_specs=[
        pl.BlockSpec((2, B, 64), lambda i: (0, i, 0)),
        pl.BlockSpec((2, B, 16), lambda i: (0, i, 0)),
    ],
    out_specs=(
        pl.BlockSpec((B, 128), lambda i: (i, 0)),
        pl.BlockSpec((2, B, 64), lambda i: (0, i, 0)),
    ),
    out_shape=(
        jax.ShapeDtypeStruct((N, 128), jnp.float32),
        jax.ShapeDtypeStruct((NC, N, 64), jnp.float32),
    ),
)


def _combk_body(p_ref, tprev_ref, degp_ref, t_ref, u_ref):
    di = _dinv(degp_ref[...])
    tk = (-2.0 * di * jnp.concatenate([p_ref[0], p_ref[1]], axis=1)
          - tprev_ref[...])
    t_ref[...] = tk
    _split64(u_ref, di * tk)


_combk = pl.pallas_call(
    _combk_body,
    grid=GRID,
    in_specs=[
        pl.BlockSpec((2, B, 64), lambda i: (0, i, 0)),
        pl.BlockSpec((B, 128), lambda i: (i, 0)),
        pl.BlockSpec((2, B, 16), lambda i: (0, i, 0)),
    ],
    out_specs=(
        pl.BlockSpec((B, 128), lambda i: (i, 0)),
        pl.BlockSpec((2, B, 64), lambda i: (0, i, 0)),
    ),
    out_shape=(
        jax.ShapeDtypeStruct((N, 128), jnp.float32),
        jax.ShapeDtypeStruct((NC, N, 64), jnp.float32),
    ),
)


def _mm1_body(t0, t1, t2, t3, t4, degp_ref, w1_ref, b1_ref, w2_ref, y_ref,
              u4_ref):
    acc = jnp.dot(t0[...], w1_ref[0], preferred_element_type=jnp.float32)
    acc += jnp.dot(t1[...], w1_ref[1], preferred_element_type=jnp.float32)
    acc += jnp.dot(t2[...], w1_ref[2], preferred_element_type=jnp.float32)
    acc += jnp.dot(t3[...], w1_ref[3], preferred_element_type=jnp.float32)
    acc += jnp.dot(t4[...], w1_ref[4], preferred_element_type=jnp.float32)
    h = _leaky(acc + b1_ref[...], ALPHA)
    y = jnp.dot(h, w2_ref[...], preferred_element_type=jnp.float32)
    for k in range(5):
        y_ref[k] = y[:, 64 * k:64 * (k + 1)]
    u4_ref[...] = _dinv(degp_ref[...]) * y[:, 256:320]


_mm1 = pl.pallas_call(
    _mm1_body,
    grid=GRID,
    in_specs=[
        pl.BlockSpec((B, 128), lambda i: (i, 0)),
        pl.BlockSpec((B, 128), lambda i: (i, 0)),
        pl.BlockSpec((B, 128), lambda i: (i, 0)),
        pl.BlockSpec((B, 128), lambda i: (i, 0)),
        pl.BlockSpec((B, 128), lambda i: (i, 0)),
        pl.BlockSpec((2, B, 16), lambda i: (0, i, 0)),
        pl.BlockSpec((5, 128, 256), lambda i: (0, 0, 0)),
        pl.BlockSpec((1, 256), lambda i: (0, 0)),
        pl.BlockSpec((256, 320), lambda i: (0, 0)),
    ],
    out_specs=(
        pl.BlockSpec((5, B, 64), lambda i: (0, i, 0)),
        pl.BlockSpec((B, 64), lambda i: (i, 0)),
    ),
    out_shape=(
        jax.ShapeDtypeStruct((5, N, 64), jnp.float32),
        jax.ShapeDtypeStruct((N, 64), jnp.float32),
    ),
)


def _make_clen(kcol, first):
    def body(p_ref, y_ref, *rest):
        if first:
            degp_ref, c_ref, u_ref = rest
            prev = 0.0
        else:
            cprev_ref, degp_ref, c_ref, u_ref = rest
            prev = cprev_ref[...]
        di = _dinv(degp_ref[...])
        ck = y_ref[0] - 2.0 * di * (p_ref[0] + p_ref[1]) - prev
        c_ref[...] = ck
        u_ref[...] = di * ck

    in_specs = [
        pl.BlockSpec((2, B, 64), lambda i: (0, i, 0)),
        pl.BlockSpec((1, B, 64), lambda i, k=kcol: (k, i, 0)),
    ]
    if not first:
        in_specs.append(pl.BlockSpec((B, 64), lambda i: (i, 0)))
    in_specs.append(pl.BlockSpec((2, B, 16), lambda i: (0, i, 0)))
    return pl.pallas_call(
        body,
        grid=GRID,
        in_specs=in_specs,
        out_specs=(
            pl.BlockSpec((B, 64), lambda i: (i, 0)),
            pl.BlockSpec((B, 64), lambda i: (i, 0)),
        ),
        out_shape=(
            jax.ShapeDtypeStruct((N, 64), jnp.float32),
            jax.ShapeDtypeStruct((N, 64), jnp.float32),
        ),
    )


_clen3 = _make_clen(3, True)
_clen2 = _make_clen(2, False)
_clen1 = _make_clen(1, False)


def _final_body(p_ref, y_ref, c2_ref, degp_ref, b2_ref, wm1_ref, bm1_ref,
                wm2_ref, bm2_ref, o_ref):
    di = _dinv(degp_ref[...])
    s = y_ref[...] - di * (p_ref[0] + p_ref[1]) - c2_ref[...] + b2_ref[...]
    t = jnp.dot(s, wm1_ref[...], preferred_element_type=jnp.float32)
    t = _leaky(t + bm1_ref[...], 0.01)
    t = jnp.dot(t, wm2_ref[...], preferred_element_type=jnp.float32)
    t = _leaky(t + bm2_ref[...], 0.01)
    t = _leaky(t, ALPHA)
    m = jnp.max(t, axis=1, keepdims=True)
    e = jnp.exp(t - m)
    lse = jnp.log(jnp.sum(e, axis=1, keepdims=True))
    o_ref[...] = t - m - lse


_final = pl.pallas_call(
    _final_body,
    grid=GRID,
    in_specs=[
        pl.BlockSpec((2, B, 64), lambda i: (0, i, 0)),
        pl.BlockSpec((B, 64), lambda i: (i, 0)),
        pl.BlockSpec((B, 64), lambda i: (i, 0)),
        pl.BlockSpec((2, B, 16), lambda i: (0, i, 0)),
        pl.BlockSpec((1, 64), lambda i: (0, 0)),
        pl.BlockSpec((64, 128), lambda i: (0, 0)),
        pl.BlockSpec((1, 128), lambda i: (0, 0)),
        pl.BlockSpec((128, 16), lambda i: (0, 0)),
        pl.BlockSpec((1, 16), lambda i: (0, 0)),
    ],
    out_specs=pl.BlockSpec((B, 16), lambda i: (i, 0)),
    out_shape=jax.ShapeDtypeStruct((N, 16), jnp.float32),
)

def kernel(x, edge_index, W1, b1, W2, b2, Wm1, bm1, Wm2, bm2):
    _app128 = _make_sc_app(True)
    _app64 = _make_sc_app(False)
    _sc_deg = _make_sc_deg()
    src = edge_index[0]
    dst = edge_index[1]
    npad = NW * EPT - E
    srcp = jnp.concatenate([src, jnp.zeros((npad,), jnp.int32)])
    srcp = srcp.reshape(NC, NS, NCHUNK, CH)
    dstp = jnp.concatenate([dst, jnp.full((npad,), TRASH, jnp.int32)])
    dstp = dstp.reshape(NC, NS, NCHUNK, CH)
    srcf = jnp.broadcast_to(
        jnp.concatenate([src, jnp.zeros((npad,), jnp.int32)]).reshape(
            1, NS, NC * NCHUNK, CH), (NC, NS, NC * NCHUNK, CH))
    dstf = jnp.broadcast_to(
        jnp.concatenate([dst, jnp.full((npad,), TRASH, jnp.int32)]).reshape(
            1, NS, NC * NCHUNK, CH), (NC, NS, NC * NCHUNK, CH))

    degp = _sc_deg(dstp)

    xs = _scale0(x, degp)
    p = _app128(xs, srcf, dstf)
    T1, u1 = _comb1(p, degp)
    p = _app128(u1, srcf, dstf)
    T2, u2 = _combk(p, x, degp)
    p = _app128(u2, srcf, dstf)
    T3, u3 = _combk(p, T1, degp)
    p = _app128(u3, srcf, dstf)
    T4, _ = _combk(p, T2, degp)

    w1cat = W1.reshape(5, 128, 256)
    b1r = b1.reshape(1, 256)
    w2cat = jnp.transpose(W2, (1, 0, 2)).reshape(256, 320)
    Y, u4 = _mm1(x, T1, T2, T3, T4, degp, w1cat, b1r, w2cat)

    p = _app64(u4, srcp, dstp)
    c3, u3b = _clen3(p, Y, degp)
    p = _app64(u3b, srcp, dstp)
    c2v, u2b = _clen2(p, Y, Y[4], degp)
    p = _app64(u2b, srcp, dstp)
    c1v, u1b = _clen1(p, Y, c3, degp)
    p = _app64(u1b, srcp, dstp)

    return _final(p, Y[0], c2v, degp, b2.reshape(1, 64),
                  Wm1, bm1.reshape(1, 128), Wm2, bm2.reshape(1, 16))


# speedup vs baseline: 1.1584x; 1.1584x over previous
import functools

import jax
import jax.numpy as jnp
from jax import lax
from jax.experimental import pallas as pl
from jax.experimental.pallas import tpu as pltpu
from jax.experimental.pallas import tpu_sc as plsc

N = 10000
E = 320000
ALPHA = 0.2

NC = 2
NS = 16
NW = NC * NS
EPT = 10240
CH = 128
NCHUNK = EPT // CH
NPAD = 10240
TRASH = N
ZROWS = NPAD // NS
DRAIN_A = 624
DRAIN_LAST = N - 15 * DRAIN_A


def _drain(acc, out2d, s):
    @pl.when(s < NS - 1)
    def _():
        pltpu.sync_copy(acc.at[pl.ds(s * DRAIN_A, DRAIN_A)],
                        out2d.at[pl.ds(s * DRAIN_A, DRAIN_A)])

    @pl.when(s == NS - 1)
    def _():
        pltpu.sync_copy(acc.at[pl.ds(15 * DRAIN_A, DRAIN_LAST)],
                        out2d.at[pl.ds(15 * DRAIN_A, DRAIN_LAST)])

@functools.cache
def _mesh():
    return plsc.VectorSubcoreMesh(core_axis_name="c", subcore_axis_name="s",
                                  num_cores=NC, num_subcores=NS)


@functools.cache
def _make_sc_app(feature_split):
    w = 64
    ept = EPT * NC if feature_split else EPT
    nchunk = ept // CH
    BLK = 20
    nblk = nchunk // BLK
    R = 4

    @functools.partial(
        pl.kernel,
        out_type=jax.ShapeDtypeStruct((NC, N, w), jnp.float32),
        mesh=_mesh(),
        compiler_params=pltpu.CompilerParams(use_tc_tiling_on_sc=False),
        scratch_types=[
            pltpu.VMEM_SHARED((NPAD, w), jnp.float32),
            pltpu.VMEM_SHARED((N, w), jnp.float32),
            pltpu.VMEM((BLK, CH), jnp.int32),
            pltpu.VMEM((BLK, CH), jnp.int32),
            pltpu.VMEM((BLK, CH), jnp.int32),
            pltpu.VMEM((BLK, CH), jnp.int32),
            [pltpu.VMEM((CH, w), jnp.float32)] * R,
            [pltpu.SemaphoreType.DMA] * R,
            [pltpu.SemaphoreType.DMA] * R,
            pltpu.SemaphoreType.DMA,
            pltpu.SemaphoreType.DMA,
            pltpu.SemaphoreType.DMA,
        ],
    )
    def app(t_hbm, src_hbm, dst_hbm, p_hbm, acc, stage, sA, dA, sB, dB,
            bufs, gs, ss, isA, isB, sts):
        c = lax.axis_index("c")
        s = lax.axis_index("s")
        src_rows = src_hbm.at[c].at[s]
        dst_rows = dst_hbm.at[c].at[s]
        tsrc = t_hbm.at[c] if feature_split else t_hbm
        buf0 = bufs[0]

        @pl.when(s < NS - 1)
        def _():
            pltpu.async_copy(tsrc.at[pl.ds(s * DRAIN_A, DRAIN_A)],
                             stage.at[pl.ds(s * DRAIN_A, DRAIN_A)], sts)

        @pl.when(s == NS - 1)
        def _():
            pltpu.async_copy(tsrc.at[pl.ds(15 * DRAIN_A, DRAIN_LAST)],
                             stage.at[pl.ds(15 * DRAIN_A, DRAIN_LAST)], sts)

        @pl.loop(0, CH)
        def _(r):
            for j in range(w // 16):
                buf0[r, pl.ds(j * 16, 16)] = jnp.zeros((16,), jnp.float32)

        @pl.loop(0, ZROWS // CH)
        def _(z):
            pltpu.sync_copy(buf0, acc.at[pl.ds(s * ZROWS + z * CH, CH)])

        @pl.when(s < NS - 1)
        def _():
            pltpu.make_async_copy(tsrc.at[pl.ds(s * DRAIN_A, DRAIN_A)],
                                  stage.at[pl.ds(s * DRAIN_A, DRAIN_A)],
                                  sts).wait()

        @pl.when(s == NS - 1)
        def _():
            pltpu.make_async_copy(tsrc.at[pl.ds(15 * DRAIN_A, DRAIN_LAST)],
                                  stage.at[pl.ds(15 * DRAIN_A, DRAIN_LAST)],
                                  sts).wait()

        plsc.subcore_barrier()

        def load_idx(blk, sref, dref, isem):
            pltpu.async_copy(src_rows.at[pl.ds(blk * BLK, BLK)], sref, isem)
            pltpu.async_copy(dst_rows.at[pl.ds(blk * BLK, BLK)], dref, isem)

        def wait_idx(sref, dref, isem):
            pltpu.make_async_copy(src_rows.at[pl.ds(0, BLK)], sref,
                                  isem).wait()
            pltpu.make_async_copy(dst_rows.at[pl.ds(0, BLK)], dref,
                                  isem).wait()

        def wait_gather(j, sref, r):
            pltpu.make_async_copy(stage.at[sref.at[r]], bufs[j],
                                  gs[j]).wait()

        def wait_scatter(j, dref, r):
            pltpu.make_async_copy(bufs[j], acc.at[dref.at[r]], ss[j]).wait()

        load_idx(0, sA, dA, isA)
        load_idx(1, sB, dB, isB)
        wait_idx(sA, dA, isA)
        for j in range(R):
            pltpu.async_copy(stage.at[sA.at[j]], bufs[j], gs[j])

        @pl.loop(0, nblk, step=2)
        def _(b):
            for ch in range(2 * BLK):
                j = ch % R
                sref, dref = (sA, dA) if ch < BLK else (sB, dB)
                r = ch % BLK
                wait_gather(j, sref, r)
                pltpu.async_copy(bufs[j], acc.at[dref.at[r]], ss[j],
                                 add=True)
                if ch == BLK - 1:
                    @pl.when(b + 2 < nblk)
                    def _():
                        load_idx(b + 2, sA, dA, isA)
                if ch == 2 * BLK - 1:
                    @pl.when(b + 3 < nblk)
                    def _():
                        load_idx(b + 3, sB, dB, isB)
                if ch == BLK - R:
                    wait_idx(sB, dB, isB)
                if ch == 2 * BLK - R:
                    @pl.when(b + 2 < nblk)
                    def _():
                        wait_idx(sA, dA, isA)
                wait_scatter(j, dref, r)
                chb = ch + R
                if chb < BLK:
                    pltpu.async_copy(stage.at[sA.at[chb]], bufs[j], gs[j])
                elif chb < 2 * BLK:
                    pltpu.async_copy(stage.at[sB.at[chb - BLK]], bufs[j],
                                     gs[j])
                else:
                    @pl.when(b + 2 < nblk)
                    def _():
                        pltpu.async_copy(stage.at[sA.at[chb - 2 * BLK]],
                                         bufs[j], gs[j])

        plsc.subcore_barrier()
        _drain(acc, p_hbm.at[c], s)

    return app


@functools.cache
def _make_sc_deg():
    @functools.partial(
        pl.kernel,
        out_type=jax.ShapeDtypeStruct((NC, N, 16), jnp.float32),
        mesh=_mesh(),
        compiler_params=pltpu.CompilerParams(use_tc_tiling_on_sc=False),
        scratch_types=[
            pltpu.VMEM_SHARED((NPAD, 16), jnp.float32),
            pltpu.VMEM((NCHUNK, CH), jnp.int32),
            pltpu.VMEM((CH, 16), jnp.float32),
            pltpu.SemaphoreType.DMA,
        ],
    )
    def _sc_deg(dst_hbm, out_hbm, acc, dst_v, ones_v, ssem):
        c = lax.axis_index("c")
        s = lax.axis_index("s")

        @pl.loop(0, CH)
        def _(r):
            ones_v[r, pl.ds(0, 16)] = jnp.zeros((16,), jnp.float32)

        @pl.loop(0, ZROWS // CH)
        def _(z):
            pltpu.sync_copy(ones_v, acc.at[pl.ds(s * ZROWS + z * CH, CH)])

        @pl.loop(0, CH)
        def _(r):
            ones_v[r, pl.ds(0, 16)] = jnp.ones((16,), jnp.float32)

        pltpu.sync_copy(dst_hbm.at[c].at[s], dst_v)
        plsc.subcore_barrier()

        @pl.loop(0, NCHUNK, step=8)
        def _(ch):
            for j in range(8):
                pltpu.async_copy(ones_v, acc.at[dst_v.at[ch + j]], ssem,
                                 add=True)
            for j in range(8):
                pltpu.make_async_copy(ones_v, acc.at[dst_v.at[ch + j]],
                                      ssem).wait()

        plsc.subcore_barrier()
        _drain(acc, out_hbm.at[c], s)

    return _sc_deg


B = 2000
GRID = (N // B,)


def _dinv(degp):
    deg = degp[0, :, 0] + degp[1, :, 0]
    return jnp.where(deg > 0, 1.0 / jnp.sqrt(jnp.maximum(deg, 1.0)),
                     0.0)[:, None]


def _leaky(x, a):
    return jnp.where(x >= 0, x, a * x)


def _split64(o_ref, v):
    o_ref[0] = v[:, 0:64]
    o_ref[1] = v[:, 64:128]


def _scale0_body(x_ref, degp_ref, o_ref):
    _split64(o_ref, _dinv(degp_ref[...]) * x_ref[...])


_scale0 = pl.pallas_call(
    _scale0_body,
    grid=GRID,
    in_specs=[
        pl.BlockSpec((B, 128), lambda i: (i, 0)),
        pl.BlockSpec((2, B, 16), lambda i: (0, i, 0)),
    ],
    out_specs=pl.BlockSpec((2, B, 64), lambda i: (0, i, 0)),
    out_shape=jax.ShapeDtypeStruct((NC, N, 64), jnp.float32),
)


def _comb1_body(p_ref, degp_ref, t_ref, u_ref):
    di = _dinv(degp_ref[...])
    t1 = -(di * jnp.concatenate([p_ref[0], p_ref[1]], axis=1))
    t_ref[...] = t1
    _split64(u_ref, di * t1)


_comb1 = pl.pallas_call(
    _comb1_body,
    grid=GRID,
    in_specs=[
        pl.BlockSpec((2, B, 64), lambda i: (0, i, 0)),
        pl.BlockSpec((2, B, 16), lambda i: (0, i, 0)),
    ],
    out_specs=(
        pl.BlockSpec((B, 128), lambda i: (i, 0)),
        pl.BlockSpec((2, B, 64), lambda i: (0, i, 0)),
    ),
    out_shape=(
        jax.ShapeDtypeStruct((N, 128), jnp.float32),
        jax.ShapeDtypeStruct((NC, N, 64), jnp.float32),
    ),
)


def _combk_body(p_ref, tprev_ref, degp_ref, t_ref, u_ref):
    di = _dinv(degp_ref[...])
    tk = (-2.0 * di * jnp.concatenate([p_ref[0], p_ref[1]], axis=1)
          - tprev_ref[...])
    t_ref[...] = tk
    _split64(u_ref, di * tk)


_combk = pl.pallas_call(
    _combk_body,
    grid=GRID,
    in_specs=[
        pl.BlockSpec((2, B, 64), lambda i: (0, i, 0)),
        pl.BlockSpec((B, 128), lambda i: (i, 0)),
        pl.BlockSpec((2, B, 16), lambda i: (0, i, 0)),
    ],
    out_specs=(
        pl.BlockSpec((B, 128), lambda i: (i, 0)),
        pl.BlockSpec((2, B, 64), lambda i: (0, i, 0)),
    ),
    out_shape=(
        jax.ShapeDtypeStruct((N, 128), jnp.float32),
        jax.ShapeDtypeStruct((NC, N, 64), jnp.float32),
    ),
)


def _mm1_body(t0, t1, t2, t3, t4, degp_ref, w1_ref, b1_ref, w2_ref, y_ref,
              u4_ref):
    acc = jnp.dot(t0[...], w1_ref[0], preferred_element_type=jnp.float32)
    acc += jnp.dot(t1[...], w1_ref[1], preferred_element_type=jnp.float32)
    acc += jnp.dot(t2[...], w1_ref[2], preferred_element_type=jnp.float32)
    acc += jnp.dot(t3[...], w1_ref[3], preferred_element_type=jnp.float32)
    acc += jnp.dot(t4[...], w1_ref[4], preferred_element_type=jnp.float32)
    h = _leaky(acc + b1_ref[...], ALPHA)
    y = jnp.dot(h, w2_ref[...], preferred_element_type=jnp.float32)
    for k in range(5):
        y_ref[k] = y[:, 64 * k:64 * (k + 1)]
    u4_ref[...] = _dinv(degp_ref[...]) * y[:, 256:320]


_mm1 = pl.pallas_call(
    _mm1_body,
    grid=GRID,
    in_specs=[
        pl.BlockSpec((B, 128), lambda i: (i, 0)),
        pl.BlockSpec((B, 128), lambda i: (i, 0)),
        pl.BlockSpec((B, 128), lambda i: (i, 0)),
        pl.BlockSpec((B, 128), lambda i: (i, 0)),
        pl.BlockSpec((B, 128), lambda i: (i, 0)),
        pl.BlockSpec((2, B, 16), lambda i: (0, i, 0)),
        pl.BlockSpec((5, 128, 256), lambda i: (0, 0, 0)),
        pl.BlockSpec((1, 256), lambda i: (0, 0)),
        pl.BlockSpec((256, 320), lambda i: (0, 0)),
    ],
    out_specs=(
        pl.BlockSpec((5, B, 64), lambda i: (0, i, 0)),
        pl.BlockSpec((B, 64), lambda i: (i, 0)),
    ),
    out_shape=(
        jax.ShapeDtypeStruct((5, N, 64), jnp.float32),
        jax.ShapeDtypeStruct((N, 64), jnp.float32),
    ),
)


def _make_clen(kcol, first):
    def body(p_ref, y_ref, *rest):
        if first:
            degp_ref, c_ref, u_ref = rest
            prev = 0.0
        else:
            cprev_ref, degp_ref, c_ref, u_ref = rest
            prev = cprev_ref[...]
        di = _dinv(degp_ref[...])
        ck = y_ref[0] - 2.0 * di * (p_ref[0] + p_ref[1]) - prev
        c_ref[...] = ck
        u_ref[...] = di * ck

    in_specs = [
        pl.BlockSpec((2, B, 64), lambda i: (0, i, 0)),
        pl.BlockSpec((1, B, 64), lambda i, k=kcol: (k, i, 0)),
    ]
    if not first:
        in_specs.append(pl.BlockSpec((B, 64), lambda i: (i, 0)))
    in_specs.append(pl.BlockSpec((2, B, 16), lambda i: (0, i, 0)))
    return pl.pallas_call(
        body,
        grid=GRID,
        in_specs=in_specs,
        out_specs=(
            pl.BlockSpec((B, 64), lambda i: (i, 0)),
            pl.BlockSpec((B, 64), lambda i: (i, 0)),
        ),
        out_shape=(
            jax.ShapeDtypeStruct((N, 64), jnp.float32),
            jax.ShapeDtypeStruct((N, 64), jnp.float32),
        ),
    )


_clen3 = _make_clen(3, True)
_clen2 = _make_clen(2, False)
_clen1 = _make_clen(1, False)


def _final_body(p_ref, y_ref, c2_ref, degp_ref, b2_ref, wm1_ref, bm1_ref,
                wm2_ref, bm2_ref, o_ref):
    di = _dinv(degp_ref[...])
    s = y_ref[...] - di * (p_ref[0] + p_ref[1]) - c2_ref[...] + b2_ref[...]
    t = jnp.dot(s, wm1_ref[...], preferred_element_type=jnp.float32)
    t = _leaky(t + bm1_ref[...], 0.01)
    t = jnp.dot(t, wm2_ref[...], preferred_element_type=jnp.float32)
    t = _leaky(t + bm2_ref[...], 0.01)
    t = _leaky(t, ALPHA)
    m = jnp.max(t, axis=1, keepdims=True)
    e = jnp.exp(t - m)
    lse = jnp.log(jnp.sum(e, axis=1, keepdims=True))
    o_ref[...] = t - m - lse


_final = pl.pallas_call(
    _final_body,
    grid=GRID,
    in_specs=[
        pl.BlockSpec((2, B, 64), lambda i: (0, i, 0)),
        pl.BlockSpec((B, 64), lambda i: (i, 0)),
        pl.BlockSpec((B, 64), lambda i: (i, 0)),
        pl.BlockSpec((2, B, 16), lambda i: (0, i, 0)),
        pl.BlockSpec((1, 64), lambda i: (0, 0)),
        pl.BlockSpec((64, 128), lambda i: (0, 0)),
        pl.BlockSpec((1, 128), lambda i: (0, 0)),
        pl.BlockSpec((128, 16), lambda i: (0, 0)),
        pl.BlockSpec((1, 16), lambda i: (0, 0)),
    ],
    out_specs=pl.BlockSpec((B, 16), lambda i: (i, 0)),
    out_shape=jax.ShapeDtypeStruct((N, 16), jnp.float32),
)

def kernel(x, edge_index, W1, b1, W2, b2, Wm1, bm1, Wm2, bm2):
    _app128 = _make_sc_app(True)
    _app64 = _make_sc_app(False)
    _sc_deg = _make_sc_deg()
    src = edge_index[0]
    dst = edge_index[1]
    npad = NW * EPT - E
    srcp = jnp.concatenate([src, jnp.zeros((npad,), jnp.int32)])
    srcp = srcp.reshape(NC, NS, NCHUNK, CH)
    dstp = jnp.concatenate([dst, jnp.full((npad,), TRASH, jnp.int32)])
    dstp = dstp.reshape(NC, NS, NCHUNK, CH)
    srcf = jnp.broadcast_to(
        jnp.concatenate([src, jnp.zeros((npad,), jnp.int32)]).reshape(
            1, NS, NC * NCHUNK, CH), (NC, NS, NC * NCHUNK, CH))
    dstf = jnp.broadcast_to(
        jnp.concatenate([dst, jnp.full((npad,), TRASH, jnp.int32)]).reshape(
            1, NS, NC * NCHUNK, CH), (NC, NS, NC * NCHUNK, CH))

    degp = _sc_deg(dstp)

    xs = _scale0(x, degp)
    p = _app128(xs, srcf, dstf)
    T1, u1 = _comb1(p, degp)
    p = _app128(u1, srcf, dstf)
    T2, u2 = _combk(p, x, degp)
    p = _app128(u2, srcf, dstf)
    T3, u3 = _combk(p, T1, degp)
    p = _app128(u3, srcf, dstf)
    T4, _ = _combk(p, T2, degp)

    w1cat = W1.reshape(5, 128, 256)
    b1r = b1.reshape(1, 256)
    w2cat = jnp.transpose(W2, (1, 0, 2)).reshape(256, 320)
    Y, u4 = _mm1(x, T1, T2, T3, T4, degp, w1cat, b1r, w2cat)

    p = _app64(u4, srcp, dstp)
    c3, u3b = _clen3(p, Y, degp)
    p = _app64(u3b, srcp, dstp)
    c2v, u2b = _clen2(p, Y, Y[4], degp)
    p = _app64(u2b, srcp, dstp)
    c1v, u1b = _clen1(p, Y, c3, degp)
    p = _app64(u1b, srcp, dstp)

    return _final(p, Y[0], c2v, degp, b2.reshape(1, 64),
                  Wm1, bm1.reshape(1, 128), Wm2, bm2.reshape(1, 16))


# speedup vs baseline: 1.1668x; 1.0072x over previous
import functools

import jax
import jax.numpy as jnp
from jax import lax
from jax.experimental import pallas as pl
from jax.experimental.pallas import tpu as pltpu
from jax.experimental.pallas import tpu_sc as plsc

N = 10000
E = 320000
ALPHA = 0.2

NC = 2
NS = 16
NW = NC * NS
EPT = 10240
CH = 128
NCHUNK = EPT // CH
NPAD = 10240
TRASH = N
ZROWS = NPAD // NS
DRAIN_A = 624
DRAIN_LAST = N - 15 * DRAIN_A


def _drain(acc, out2d, s):
    @pl.when(s < NS - 1)
    def _():
        pltpu.sync_copy(acc.at[pl.ds(s * DRAIN_A, DRAIN_A)],
                        out2d.at[pl.ds(s * DRAIN_A, DRAIN_A)])

    @pl.when(s == NS - 1)
    def _():
        pltpu.sync_copy(acc.at[pl.ds(15 * DRAIN_A, DRAIN_LAST)],
                        out2d.at[pl.ds(15 * DRAIN_A, DRAIN_LAST)])

@functools.cache
def _mesh():
    return plsc.VectorSubcoreMesh(core_axis_name="c", subcore_axis_name="s",
                                  num_cores=NC, num_subcores=NS)


@functools.cache
def _make_sc_app(feature_split):
    w = 64
    ept = EPT * NC if feature_split else EPT
    nchunk = ept // CH
    BLK = 20
    nblk = nchunk // BLK
    R = 4

    @functools.partial(
        pl.kernel,
        out_type=jax.ShapeDtypeStruct((NC, N, w), jnp.float32),
        mesh=_mesh(),
        compiler_params=pltpu.CompilerParams(use_tc_tiling_on_sc=False),
        scratch_types=[
            pltpu.VMEM_SHARED((NPAD, w), jnp.float32),
            pltpu.VMEM_SHARED((N, w), jnp.float32),
            pltpu.VMEM((BLK, CH), jnp.int32),
            pltpu.VMEM((BLK, CH), jnp.int32),
            pltpu.VMEM((BLK, CH), jnp.int32),
            pltpu.VMEM((BLK, CH), jnp.int32),
            [pltpu.VMEM((CH, w), jnp.float32)] * R,
            [pltpu.SemaphoreType.DMA] * R,
            [pltpu.SemaphoreType.DMA] * R,
            pltpu.SemaphoreType.DMA,
            pltpu.SemaphoreType.DMA,
            pltpu.SemaphoreType.DMA,
        ],
    )
    def app(t_hbm, src_hbm, dst_hbm, p_hbm, acc, stage, sA, dA, sB, dB,
            bufs, gs, ss, isA, isB, sts):
        c = lax.axis_index("c")
        s = lax.axis_index("s")
        src_rows = src_hbm.at[c].at[s]
        dst_rows = dst_hbm.at[c].at[s]
        tsrc = t_hbm.at[c] if feature_split else t_hbm
        buf0 = bufs[0]

        @pl.when(s < NS - 1)
        def _():
            pltpu.async_copy(tsrc.at[pl.ds(s * DRAIN_A, DRAIN_A)],
                             stage.at[pl.ds(s * DRAIN_A, DRAIN_A)], sts)

        @pl.when(s == NS - 1)
        def _():
            pltpu.async_copy(tsrc.at[pl.ds(15 * DRAIN_A, DRAIN_LAST)],
                             stage.at[pl.ds(15 * DRAIN_A, DRAIN_LAST)], sts)

        @pl.loop(0, CH)
        def _(r):
            for j in range(w // 16):
                buf0[r, pl.ds(j * 16, 16)] = jnp.zeros((16,), jnp.float32)

        nz = ZROWS // CH
        for z in range(nz):
            pltpu.async_copy(buf0, acc.at[pl.ds(s * ZROWS + z * CH, CH)],
                             ss[z % R])
        for z in range(nz):
            pltpu.make_async_copy(buf0,
                                  acc.at[pl.ds(s * ZROWS + z * CH, CH)],
                                  ss[z % R]).wait()

        @pl.when(s < NS - 1)
        def _():
            pltpu.make_async_copy(tsrc.at[pl.ds(s * DRAIN_A, DRAIN_A)],
                                  stage.at[pl.ds(s * DRAIN_A, DRAIN_A)],
                                  sts).wait()

        @pl.when(s == NS - 1)
        def _():
            pltpu.make_async_copy(tsrc.at[pl.ds(15 * DRAIN_A, DRAIN_LAST)],
                                  stage.at[pl.ds(15 * DRAIN_A, DRAIN_LAST)],
                                  sts).wait()

        plsc.subcore_barrier()

        def load_idx(blk, sref, dref, isem):
            pltpu.async_copy(src_rows.at[pl.ds(blk * BLK, BLK)], sref, isem)
            pltpu.async_copy(dst_rows.at[pl.ds(blk * BLK, BLK)], dref, isem)

        def wait_idx(sref, dref, isem):
            pltpu.make_async_copy(src_rows.at[pl.ds(0, BLK)], sref,
                                  isem).wait()
            pltpu.make_async_copy(dst_rows.at[pl.ds(0, BLK)], dref,
                                  isem).wait()

        def wait_gather(j, sref, r):
            pltpu.make_async_copy(stage.at[sref.at[r]], bufs[j],
                                  gs[j]).wait()

        def wait_scatter(j, dref, r):
            pltpu.make_async_copy(bufs[j], acc.at[dref.at[r]], ss[j]).wait()

        load_idx(0, sA, dA, isA)
        load_idx(1, sB, dB, isB)
        wait_idx(sA, dA, isA)
        for j in range(R):
            pltpu.async_copy(stage.at[sA.at[j]], bufs[j], gs[j])

        @pl.loop(0, nblk, step=2)
        def _(b):
            for ch in range(2 * BLK):
                j = ch % R
                sref, dref = (sA, dA) if ch < BLK else (sB, dB)
                r = ch % BLK
                wait_gather(j, sref, r)
                pltpu.async_copy(bufs[j], acc.at[dref.at[r]], ss[j],
                                 add=True)
                if ch == BLK - 1:
                    @pl.when(b + 2 < nblk)
                    def _():
                        load_idx(b + 2, sA, dA, isA)
                if ch == 2 * BLK - 1:
                    @pl.when(b + 3 < nblk)
                    def _():
                        load_idx(b + 3, sB, dB, isB)
                if ch == BLK - R:
                    wait_idx(sB, dB, isB)
                if ch == 2 * BLK - R:
                    @pl.when(b + 2 < nblk)
                    def _():
                        wait_idx(sA, dA, isA)
                wait_scatter(j, dref, r)
                chb = ch + R
                if chb < BLK:
                    pltpu.async_copy(stage.at[sA.at[chb]], bufs[j], gs[j])
                elif chb < 2 * BLK:
                    pltpu.async_copy(stage.at[sB.at[chb - BLK]], bufs[j],
                                     gs[j])
                else:
                    @pl.when(b + 2 < nblk)
                    def _():
                        pltpu.async_copy(stage.at[sA.at[chb - 2 * BLK]],
                                         bufs[j], gs[j])

        plsc.subcore_barrier()
        _drain(acc, p_hbm.at[c], s)

    return app


@functools.cache
def _make_sc_deg():
    @functools.partial(
        pl.kernel,
        out_type=jax.ShapeDtypeStruct((NC, N, 16), jnp.float32),
        mesh=_mesh(),
        compiler_params=pltpu.CompilerParams(use_tc_tiling_on_sc=False),
        scratch_types=[
            pltpu.VMEM_SHARED((NPAD, 16), jnp.float32),
            pltpu.VMEM((NCHUNK, CH), jnp.int32),
            pltpu.VMEM((CH, 16), jnp.float32),
            pltpu.SemaphoreType.DMA,
        ],
    )
    def _sc_deg(dst_hbm, out_hbm, acc, dst_v, ones_v, ssem):
        c = lax.axis_index("c")
        s = lax.axis_index("s")

        @pl.loop(0, CH)
        def _(r):
            ones_v[r, pl.ds(0, 16)] = jnp.zeros((16,), jnp.float32)

        @pl.loop(0, ZROWS // CH)
        def _(z):
            pltpu.sync_copy(ones_v, acc.at[pl.ds(s * ZROWS + z * CH, CH)])

        @pl.loop(0, CH)
        def _(r):
            ones_v[r, pl.ds(0, 16)] = jnp.ones((16,), jnp.float32)

        pltpu.sync_copy(dst_hbm.at[c].at[s], dst_v)
        plsc.subcore_barrier()

        @pl.loop(0, NCHUNK, step=8)
        def _(ch):
            for j in range(8):
                pltpu.async_copy(ones_v, acc.at[dst_v.at[ch + j]], ssem,
                                 add=True)
            for j in range(8):
                pltpu.make_async_copy(ones_v, acc.at[dst_v.at[ch + j]],
                                      ssem).wait()

        plsc.subcore_barrier()
        _drain(acc, out_hbm.at[c], s)

    return _sc_deg


B = 5000
GRID = (N // B,)
BM = 2000
GRIDM = (N // BM,)


def _dinv(degp):
    deg = degp[0, :, 0] + degp[1, :, 0]
    return jnp.where(deg > 0, 1.0 / jnp.sqrt(jnp.maximum(deg, 1.0)),
                     0.0)[:, None]


def _leaky(x, a):
    return jnp.where(x >= 0, x, a * x)


def _split64(o_ref, v):
    o_ref[0] = v[:, 0:64]
    o_ref[1] = v[:, 64:128]


def _scale0_body(x_ref, degp_ref, o_ref):
    _split64(o_ref, _dinv(degp_ref[...]) * x_ref[...])


_scale0 = pl.pallas_call(
    _scale0_body,
    grid=GRID,
    in_specs=[
        pl.BlockSpec((B, 128), lambda i: (i, 0)),
        pl.BlockSpec((2, B, 16), lambda i: (0, i, 0)),
    ],
    out_specs=pl.BlockSpec((2, B, 64), lambda i: (0, i, 0)),
    out_shape=jax.ShapeDtypeStruct((NC, N, 64), jnp.float32),
)


def _comb1_body(p_ref, degp_ref, t_ref, u_ref):
    di = _dinv(degp_ref[...])
    t1 = -(di * jnp.concatenate([p_ref[0], p_ref[1]], axis=1))
    t_ref[...] = t1
    _split64(u_ref, di * t1)


_comb1 = pl.pallas_call(
    _comb1_body,
    grid=GRID,
    in_specs=[
        pl.BlockSpec((2, B, 64), lambda i: (0, i, 0)),
        pl.BlockSpec((2, B, 16), lambda i: (0, i, 0)),
    ],
    out_specs=(
        pl.BlockSpec((B, 128), lambda i: (i, 0)),
        pl.BlockSpec((2, B, 64), lambda i: (0, i, 0)),
    ),
    out_shape=(
        jax.ShapeDtypeStruct((N, 128), jnp.float32),
        jax.ShapeDtypeStruct((NC, N, 64), jnp.float32),
    ),
)


def _combk_body(p_ref, tprev_ref, degp_ref, t_ref, u_ref):
    di = _dinv(degp_ref[...])
    tk = (-2.0 * di * jnp.concatenate([p_ref[0], p_ref[1]], axis=1)
          - tprev_ref[...])
    t_ref[...] = tk
    _split64(u_ref, di * tk)


_combk = pl.pallas_call(
    _combk_body,
    grid=GRID,
    in_specs=[
        pl.BlockSpec((2, B, 64), lambda i: (0, i, 0)),
        pl.BlockSpec((B, 128), lambda i: (i, 0)),
        pl.BlockSpec((2, B, 16), lambda i: (0, i, 0)),
    ],
    out_specs=(
        pl.BlockSpec((B, 128), lambda i: (i, 0)),
        pl.BlockSpec((2, B, 64), lambda i: (0, i, 0)),
    ),
    out_shape=(
        jax.ShapeDtypeStruct((N, 128), jnp.float32),
        jax.ShapeDtypeStruct((NC, N, 64), jnp.float32),
    ),
)


def _mm1_body(t0, t1, t2, t3, t4, degp_ref, w1_ref, b1_ref, w2_ref, y_ref,
              u4_ref):
    acc = jnp.dot(t0[...], w1_ref[0], preferred_element_type=jnp.float32)
    acc += jnp.dot(t1[...], w1_ref[1], preferred_element_type=jnp.float32)
    acc += jnp.dot(t2[...], w1_ref[2], preferred_element_type=jnp.float32)
    acc += jnp.dot(t3[...], w1_ref[3], preferred_element_type=jnp.float32)
    acc += jnp.dot(t4[...], w1_ref[4], preferred_element_type=jnp.float32)
    h = _leaky(acc + b1_ref[...], ALPHA)
    y = jnp.dot(h, w2_ref[...], preferred_element_type=jnp.float32)
    for k in range(5):
        y_ref[k] = y[:, 64 * k:64 * (k + 1)]
    u4_ref[...] = _dinv(degp_ref[...]) * y[:, 256:320]


_mm1 = pl.pallas_call(
    _mm1_body,
    grid=GRIDM,
    in_specs=[
        pl.BlockSpec((BM, 128), lambda i: (i, 0)),
        pl.BlockSpec((BM, 128), lambda i: (i, 0)),
        pl.BlockSpec((BM, 128), lambda i: (i, 0)),
        pl.BlockSpec((BM, 128), lambda i: (i, 0)),
        pl.BlockSpec((BM, 128), lambda i: (i, 0)),
        pl.BlockSpec((2, BM, 16), lambda i: (0, i, 0)),
        pl.BlockSpec((5, 128, 256), lambda i: (0, 0, 0)),
        pl.BlockSpec((1, 256), lambda i: (0, 0)),
        pl.BlockSpec((256, 320), lambda i: (0, 0)),
    ],
    out_specs=(
        pl.BlockSpec((5, BM, 64), lambda i: (0, i, 0)),
        pl.BlockSpec((BM, 64), lambda i: (i, 0)),
    ),
    out_shape=(
        jax.ShapeDtypeStruct((5, N, 64), jnp.float32),
        jax.ShapeDtypeStruct((N, 64), jnp.float32),
    ),
)


def _make_clen(kcol, first):
    def body(p_ref, y_ref, *rest):
        if first:
            degp_ref, c_ref, u_ref = rest
            prev = 0.0
        else:
            cprev_ref, degp_ref, c_ref, u_ref = rest
            prev = cprev_ref[...]
        di = _dinv(degp_ref[...])
        ck = y_ref[0] - 2.0 * di * (p_ref[0] + p_ref[1]) - prev
        c_ref[...] = ck
        u_ref[...] = di * ck

    in_specs = [
        pl.BlockSpec((2, B, 64), lambda i: (0, i, 0)),
        pl.BlockSpec((1, B, 64), lambda i, k=kcol: (k, i, 0)),
    ]
    if not first:
        in_specs.append(pl.BlockSpec((B, 64), lambda i: (i, 0)))
    in_specs.append(pl.BlockSpec((2, B, 16), lambda i: (0, i, 0)))
    return pl.pallas_call(
        body,
        grid=GRID,
        in_specs=in_specs,
        out_specs=(
            pl.BlockSpec((B, 64), lambda i: (i, 0)),
            pl.BlockSpec((B, 64), lambda i: (i, 0)),
        ),
        out_shape=(
            jax.ShapeDtypeStruct((N, 64), jnp.float32),
            jax.ShapeDtypeStruct((N, 64), jnp.float32),
        ),
    )


_clen3 = _make_clen(3, True)
_clen2 = _make_clen(2, False)
_clen1 = _make_clen(1, False)


def _final_body(p_ref, y_ref, c2_ref, degp_ref, b2_ref, wm1_ref, bm1_ref,
                wm2_ref, bm2_ref, o_ref):
    di = _dinv(degp_ref[...])
    s = y_ref[...] - di * (p_ref[0] + p_ref[1]) - c2_ref[...] + b2_ref[...]
    t = jnp.dot(s, wm1_ref[...], preferred_element_type=jnp.float32)
    t = _leaky(t + bm1_ref[...], 0.01)
    t = jnp.dot(t, wm2_ref[...], preferred_element_type=jnp.float32)
    t = _leaky(t + bm2_ref[...], 0.01)
    t = _leaky(t, ALPHA)
    m = jnp.max(t, axis=1, keepdims=True)
    e = jnp.exp(t - m)
    lse = jnp.log(jnp.sum(e, axis=1, keepdims=True))
    o_ref[...] = t - m - lse


_final = pl.pallas_call(
    _final_body,
    grid=GRID,
    in_specs=[
        pl.BlockSpec((2, B, 64), lambda i: (0, i, 0)),
        pl.BlockSpec((B, 64), lambda i: (i, 0)),
        pl.BlockSpec((B, 64), lambda i: (i, 0)),
        pl.BlockSpec((2, B, 16), lambda i: (0, i, 0)),
        pl.BlockSpec((1, 64), lambda i: (0, 0)),
        pl.BlockSpec((64, 128), lambda i: (0, 0)),
        pl.BlockSpec((1, 128), lambda i: (0, 0)),
        pl.BlockSpec((128, 16), lambda i: (0, 0)),
        pl.BlockSpec((1, 16), lambda i: (0, 0)),
    ],
    out_specs=pl.BlockSpec((B, 16), lambda i: (i, 0)),
    out_shape=jax.ShapeDtypeStruct((N, 16), jnp.float32),
)

def kernel(x, edge_index, W1, b1, W2, b2, Wm1, bm1, Wm2, bm2):
    _app128 = _make_sc_app(True)
    _app64 = _make_sc_app(False)
    _sc_deg = _make_sc_deg()
    src = edge_index[0]
    dst = edge_index[1]
    npad = NW * EPT - E
    srcp = jnp.concatenate([src, jnp.zeros((npad,), jnp.int32)])
    srcp = srcp.reshape(NC, NS, NCHUNK, CH)
    dstp = jnp.concatenate([dst, jnp.full((npad,), TRASH, jnp.int32)])
    dstp = dstp.reshape(NC, NS, NCHUNK, CH)
    srcf = jnp.broadcast_to(
        jnp.concatenate([src, jnp.zeros((npad,), jnp.int32)]).reshape(
            1, NS, NC * NCHUNK, CH), (NC, NS, NC * NCHUNK, CH))
    dstf = jnp.broadcast_to(
        jnp.concatenate([dst, jnp.full((npad,), TRASH, jnp.int32)]).reshape(
            1, NS, NC * NCHUNK, CH), (NC, NS, NC * NCHUNK, CH))

    degp = _sc_deg(dstp)

    xs = _scale0(x, degp)
    p = _app128(xs, srcf, dstf)
    T1, u1 = _comb1(p, degp)
    p = _app128(u1, srcf, dstf)
    T2, u2 = _combk(p, x, degp)
    p = _app128(u2, srcf, dstf)
    T3, u3 = _combk(p, T1, degp)
    p = _app128(u3, srcf, dstf)
    T4, _ = _combk(p, T2, degp)

    w1cat = W1.reshape(5, 128, 256)
    b1r = b1.reshape(1, 256)
    w2cat = jnp.transpose(W2, (1, 0, 2)).reshape(256, 320)
    Y, u4 = _mm1(x, T1, T2, T3, T4, degp, w1cat, b1r, w2cat)

    p = _app64(u4, srcp, dstp)
    c3, u3b = _clen3(p, Y, degp)
    p = _app64(u3b, srcp, dstp)
    c2v, u2b = _clen2(p, Y, Y[4], degp)
    p = _app64(u2b, srcp, dstp)
    c1v, u1b = _clen1(p, Y, c3, degp)
    p = _app64(u1b, srcp, dstp)

    return _final(p, Y[0], c2v, degp, b2.reshape(1, 64),
                  Wm1, bm1.reshape(1, 128), Wm2, bm2.reshape(1, 16))


# speedup vs baseline: 1.3503x; 1.1573x over previous
import functools

import jax
import jax.numpy as jnp
from jax import lax
from jax.experimental import pallas as pl
from jax.experimental.pallas import tpu as pltpu
from jax.experimental.pallas import tpu_sc as plsc

N = 10000
E = 320000
ALPHA = 0.2

NC = 2
NS = 16
NW = NC * NS
EPT = 10240
CH = 128
NCHUNK = EPT // CH
NPAD = 10240
TRASH = N
ZROWS = NPAD // NS
DRAIN_A = 624
DRAIN_LAST = N - 15 * DRAIN_A


def _drain(acc, out2d, s, colo, w):
    @pl.when(s < NS - 1)
    def _():
        pltpu.sync_copy(acc.at[pl.ds(s * DRAIN_A, DRAIN_A)],
                        out2d.at[pl.ds(s * DRAIN_A, DRAIN_A),
                                 pl.ds(colo, w)])

    @pl.when(s == NS - 1)
    def _():
        pltpu.sync_copy(acc.at[pl.ds(15 * DRAIN_A, DRAIN_LAST)],
                        out2d.at[pl.ds(15 * DRAIN_A, DRAIN_LAST),
                                 pl.ds(colo, w)])

@functools.cache
def _mesh():
    return plsc.VectorSubcoreMesh(core_axis_name="c", subcore_axis_name="s",
                                  num_cores=NC, num_subcores=NS)


@functools.cache
def _make_sc_app(feature_split):
    w = 64
    ept = EPT * NC if feature_split else EPT
    nchunk = ept // CH
    BLK = 20
    nblk = nchunk // BLK
    R = 4

    @functools.partial(
        pl.kernel,
        out_type=jax.ShapeDtypeStruct((N, 128), jnp.float32),
        mesh=_mesh(),
        compiler_params=pltpu.CompilerParams(use_tc_tiling_on_sc=False),
        scratch_types=[
            pltpu.VMEM_SHARED((NPAD, w), jnp.float32),
            pltpu.VMEM_SHARED((N, w), jnp.float32),
            pltpu.VMEM((BLK, CH), jnp.int32),
            pltpu.VMEM((BLK, CH), jnp.int32),
            pltpu.VMEM((BLK, CH), jnp.int32),
            pltpu.VMEM((BLK, CH), jnp.int32),
            [pltpu.VMEM((CH, w), jnp.float32)] * R,
            [pltpu.SemaphoreType.DMA] * R,
            [pltpu.SemaphoreType.DMA] * R,
            pltpu.SemaphoreType.DMA,
            pltpu.SemaphoreType.DMA,
            pltpu.SemaphoreType.DMA,
        ],
    )
    def app(t_hbm, src_hbm, dst_hbm, p_hbm, acc, stage, sA, dA, sB, dB,
            bufs, gs, ss, isA, isB, sts):
        c = lax.axis_index("c")
        s = lax.axis_index("s")
        src_rows = src_hbm.at[c].at[s]
        dst_rows = dst_hbm.at[c].at[s]
        colo = c * 64 if feature_split else c * 0
        buf0 = bufs[0]

        @pl.when(s < NS - 1)
        def _():
            pltpu.async_copy(
                t_hbm.at[pl.ds(s * DRAIN_A, DRAIN_A), pl.ds(colo, w)],
                stage.at[pl.ds(s * DRAIN_A, DRAIN_A)], sts)

        @pl.when(s == NS - 1)
        def _():
            pltpu.async_copy(
                t_hbm.at[pl.ds(15 * DRAIN_A, DRAIN_LAST), pl.ds(colo, w)],
                stage.at[pl.ds(15 * DRAIN_A, DRAIN_LAST)], sts)

        @pl.loop(0, CH)
        def _(r):
            for j in range(w // 16):
                buf0[r, pl.ds(j * 16, 16)] = jnp.zeros((16,), jnp.float32)

        nz = ZROWS // CH
        for z in range(nz):
            pltpu.async_copy(buf0, acc.at[pl.ds(s * ZROWS + z * CH, CH)],
                             ss[z % R])
        for z in range(nz):
            pltpu.make_async_copy(buf0,
                                  acc.at[pl.ds(s * ZROWS + z * CH, CH)],
                                  ss[z % R]).wait()

        @pl.when(s < NS - 1)
        def _():
            pltpu.make_async_copy(
                t_hbm.at[pl.ds(s * DRAIN_A, DRAIN_A), pl.ds(colo, w)],
                stage.at[pl.ds(s * DRAIN_A, DRAIN_A)], sts).wait()

        @pl.when(s == NS - 1)
        def _():
            pltpu.make_async_copy(
                t_hbm.at[pl.ds(15 * DRAIN_A, DRAIN_LAST), pl.ds(colo, w)],
                stage.at[pl.ds(15 * DRAIN_A, DRAIN_LAST)], sts).wait()

        plsc.subcore_barrier()

        def load_idx(blk, sref, dref, isem):
            pltpu.async_copy(src_rows.at[pl.ds(blk * BLK, BLK)], sref, isem)
            pltpu.async_copy(dst_rows.at[pl.ds(blk * BLK, BLK)], dref, isem)

        def wait_idx(sref, dref, isem):
            pltpu.make_async_copy(src_rows.at[pl.ds(0, BLK)], sref,
                                  isem).wait()
            pltpu.make_async_copy(dst_rows.at[pl.ds(0, BLK)], dref,
                                  isem).wait()

        def wait_gather(j, sref, r):
            pltpu.make_async_copy(stage.at[sref.at[r]], bufs[j],
                                  gs[j]).wait()

        def wait_scatter(j, dref, r):
            pltpu.make_async_copy(bufs[j], acc.at[dref.at[r]], ss[j]).wait()

        load_idx(0, sA, dA, isA)
        load_idx(1, sB, dB, isB)
        wait_idx(sA, dA, isA)
        for j in range(R):
            pltpu.async_copy(stage.at[sA.at[j]], bufs[j], gs[j])

        @pl.loop(0, nblk, step=2)
        def _(b):
            for ch in range(2 * BLK):
                j = ch % R
                sref, dref = (sA, dA) if ch < BLK else (sB, dB)
                r = ch % BLK
                wait_gather(j, sref, r)
                pltpu.async_copy(bufs[j], acc.at[dref.at[r]], ss[j],
                                 add=True)
                if ch == BLK - 1:
                    @pl.when(b + 2 < nblk)
                    def _():
                        load_idx(b + 2, sA, dA, isA)
                if ch == 2 * BLK - 1:
                    @pl.when(b + 3 < nblk)
                    def _():
                        load_idx(b + 3, sB, dB, isB)
                if ch == BLK - R:
                    wait_idx(sB, dB, isB)
                if ch == 2 * BLK - R:
                    @pl.when(b + 2 < nblk)
                    def _():
                        wait_idx(sA, dA, isA)
                wait_scatter(j, dref, r)
                chb = ch + R
                if chb < BLK:
                    pltpu.async_copy(stage.at[sA.at[chb]], bufs[j], gs[j])
                elif chb < 2 * BLK:
                    pltpu.async_copy(stage.at[sB.at[chb - BLK]], bufs[j],
                                     gs[j])
                else:
                    @pl.when(b + 2 < nblk)
                    def _():
                        pltpu.async_copy(stage.at[sA.at[chb - 2 * BLK]],
                                         bufs[j], gs[j])

        plsc.subcore_barrier()
        _drain(acc, p_hbm, s, c * 64, w)

    return app


@functools.cache
def _make_sc_deg():
    @functools.partial(
        pl.kernel,
        out_type=jax.ShapeDtypeStruct((N, 128), jnp.float32),
        mesh=_mesh(),
        compiler_params=pltpu.CompilerParams(use_tc_tiling_on_sc=False),
        scratch_types=[
            pltpu.VMEM_SHARED((NPAD, 16), jnp.float32),
            pltpu.VMEM((NCHUNK, CH), jnp.int32),
            pltpu.VMEM((CH, 16), jnp.float32),
            pltpu.SemaphoreType.DMA,
        ],
    )
    def _sc_deg(dst_hbm, out_hbm, acc, dst_v, ones_v, ssem):
        c = lax.axis_index("c")
        s = lax.axis_index("s")

        @pl.loop(0, CH)
        def _(r):
            ones_v[r, pl.ds(0, 16)] = jnp.zeros((16,), jnp.float32)

        @pl.loop(0, ZROWS // CH)
        def _(z):
            pltpu.sync_copy(ones_v, acc.at[pl.ds(s * ZROWS + z * CH, CH)])

        @pl.loop(0, CH)
        def _(r):
            ones_v[r, pl.ds(0, 16)] = jnp.ones((16,), jnp.float32)

        pltpu.sync_copy(dst_hbm.at[c].at[s], dst_v)
        plsc.subcore_barrier()

        @pl.loop(0, NCHUNK, step=8)
        def _(ch):
            for j in range(8):
                pltpu.async_copy(ones_v, acc.at[dst_v.at[ch + j]], ssem,
                                 add=True)
            for j in range(8):
                pltpu.make_async_copy(ones_v, acc.at[dst_v.at[ch + j]],
                                      ssem).wait()

        plsc.subcore_barrier()
        _drain(acc, out_hbm, s, c * 16, 16)

    return _sc_deg


B = 5000
GRID = (N // B,)
BM = 2000
GRIDM = (N // BM,)


def _dinv(degp):
    deg = degp[:, 0] + degp[:, 16]
    return jnp.where(deg > 0, 1.0 / jnp.sqrt(jnp.maximum(deg, 1.0)),
                     0.0)[:, None]


def _leaky(x, a):
    return jnp.where(x >= 0, x, a * x)


def _pad128(v):
    return jnp.concatenate([v, jnp.zeros_like(v)], axis=1)


def _scale0_body(x_ref, degp_ref, o_ref):
    o_ref[...] = _dinv(degp_ref[...]) * x_ref[...]


_scale0 = pl.pallas_call(
    _scale0_body,
    grid=GRID,
    in_specs=[
        pl.BlockSpec((B, 128), lambda i: (i, 0)),
        pl.BlockSpec((B, 128), lambda i: (i, 0)),
    ],
    out_specs=pl.BlockSpec((B, 128), lambda i: (i, 0)),
    out_shape=jax.ShapeDtypeStruct((N, 128), jnp.float32),
)


def _comb1_body(p_ref, degp_ref, t_ref, u_ref):
    di = _dinv(degp_ref[...])
    t1 = -(di * p_ref[...])
    t_ref[...] = t1
    u_ref[...] = di * t1


_comb1 = pl.pallas_call(
    _comb1_body,
    grid=GRID,
    in_specs=[
        pl.BlockSpec((B, 128), lambda i: (i, 0)),
        pl.BlockSpec((B, 128), lambda i: (i, 0)),
    ],
    out_specs=(
        pl.BlockSpec((B, 128), lambda i: (i, 0)),
        pl.BlockSpec((B, 128), lambda i: (i, 0)),
    ),
    out_shape=(
        jax.ShapeDtypeStruct((N, 128), jnp.float32),
        jax.ShapeDtypeStruct((N, 128), jnp.float32),
    ),
)


def _combk_body(p_ref, tprev_ref, degp_ref, t_ref, u_ref):
    di = _dinv(degp_ref[...])
    tk = -2.0 * di * p_ref[...] - tprev_ref[...]
    t_ref[...] = tk
    u_ref[...] = di * tk


_combk = pl.pallas_call(
    _combk_body,
    grid=GRID,
    in_specs=[
        pl.BlockSpec((B, 128), lambda i: (i, 0)),
        pl.BlockSpec((B, 128), lambda i: (i, 0)),
        pl.BlockSpec((B, 128), lambda i: (i, 0)),
    ],
    out_specs=(
        pl.BlockSpec((B, 128), lambda i: (i, 0)),
        pl.BlockSpec((B, 128), lambda i: (i, 0)),
    ),
    out_shape=(
        jax.ShapeDtypeStruct((N, 128), jnp.float32),
        jax.ShapeDtypeStruct((N, 128), jnp.float32),
    ),
)


def _mm1_body(t0, t1, t2, t3, t4, degp_ref, w1_ref, b1_ref, w2_ref, y_ref,
              u4_ref):
    acc = jnp.dot(t0[...], w1_ref[0], preferred_element_type=jnp.float32)
    acc += jnp.dot(t1[...], w1_ref[1], preferred_element_type=jnp.float32)
    acc += jnp.dot(t2[...], w1_ref[2], preferred_element_type=jnp.float32)
    acc += jnp.dot(t3[...], w1_ref[3], preferred_element_type=jnp.float32)
    acc += jnp.dot(t4[...], w1_ref[4], preferred_element_type=jnp.float32)
    h = _leaky(acc + b1_ref[...], ALPHA)
    y = jnp.dot(h, w2_ref[...], preferred_element_type=jnp.float32)
    for k in range(5):
        y_ref[k] = y[:, 64 * k:64 * (k + 1)]
    u4_ref[...] = _pad128(_dinv(degp_ref[...]) * y[:, 256:320])


_mm1 = pl.pallas_call(
    _mm1_body,
    grid=GRIDM,
    in_specs=[
        pl.BlockSpec((BM, 128), lambda i: (i, 0)),
        pl.BlockSpec((BM, 128), lambda i: (i, 0)),
        pl.BlockSpec((BM, 128), lambda i: (i, 0)),
        pl.BlockSpec((BM, 128), lambda i: (i, 0)),
        pl.BlockSpec((BM, 128), lambda i: (i, 0)),
        pl.BlockSpec((BM, 128), lambda i: (i, 0)),
        pl.BlockSpec((5, 128, 256), lambda i: (0, 0, 0)),
        pl.BlockSpec((1, 256), lambda i: (0, 0)),
        pl.BlockSpec((256, 320), lambda i: (0, 0)),
    ],
    out_specs=(
        pl.BlockSpec((5, BM, 64), lambda i: (0, i, 0)),
        pl.BlockSpec((BM, 128), lambda i: (i, 0)),
    ),
    out_shape=(
        jax.ShapeDtypeStruct((5, N, 64), jnp.float32),
        jax.ShapeDtypeStruct((N, 128), jnp.float32),
    ),
)


def _make_clen(kcol, first):
    def body(p_ref, y_ref, *rest):
        if first:
            degp_ref, c_ref, u_ref = rest
            prev = 0.0
        else:
            cprev_ref, degp_ref, c_ref, u_ref = rest
            prev = cprev_ref[...]
        di = _dinv(degp_ref[...])
        psum = p_ref[:, 0:64] + p_ref[:, 64:128]
        ck = y_ref[0] - 2.0 * di * psum - prev
        c_ref[...] = ck
        u_ref[...] = _pad128(di * ck)

    in_specs = [
        pl.BlockSpec((B, 128), lambda i: (i, 0)),
        pl.BlockSpec((1, B, 64), lambda i, k=kcol: (k, i, 0)),
    ]
    if not first:
        in_specs.append(pl.BlockSpec((B, 64), lambda i: (i, 0)))
    in_specs.append(pl.BlockSpec((B, 128), lambda i: (i, 0)))
    return pl.pallas_call(
        body,
        grid=GRID,
        in_specs=in_specs,
        out_specs=(
            pl.BlockSpec((B, 64), lambda i: (i, 0)),
            pl.BlockSpec((B, 128), lambda i: (i, 0)),
        ),
        out_shape=(
            jax.ShapeDtypeStruct((N, 64), jnp.float32),
            jax.ShapeDtypeStruct((N, 128), jnp.float32),
        ),
    )


_clen3 = _make_clen(3, True)
_clen2 = _make_clen(2, False)
_clen1 = _make_clen(1, False)


def _final_body(p_ref, y_ref, c2_ref, degp_ref, b2_ref, wm1_ref, bm1_ref,
                wm2_ref, bm2_ref, o_ref):
    di = _dinv(degp_ref[...])
    psum = p_ref[:, 0:64] + p_ref[:, 64:128]
    s = y_ref[...] - di * psum - c2_ref[...] + b2_ref[...]
    t = jnp.dot(s, wm1_ref[...], preferred_element_type=jnp.float32)
    t = _leaky(t + bm1_ref[...], 0.01)
    t = jnp.dot(t, wm2_ref[...], preferred_element_type=jnp.float32)
    t = _leaky(t + bm2_ref[...], 0.01)
    t = _leaky(t, ALPHA)
    m = jnp.max(t, axis=1, keepdims=True)
    e = jnp.exp(t - m)
    lse = jnp.log(jnp.sum(e, axis=1, keepdims=True))
    o_ref[...] = t - m - lse


_final = pl.pallas_call(
    _final_body,
    grid=GRID,
    in_specs=[
        pl.BlockSpec((B, 128), lambda i: (i, 0)),
        pl.BlockSpec((B, 64), lambda i: (i, 0)),
        pl.BlockSpec((B, 64), lambda i: (i, 0)),
        pl.BlockSpec((B, 128), lambda i: (i, 0)),
        pl.BlockSpec((1, 64), lambda i: (0, 0)),
        pl.BlockSpec((64, 128), lambda i: (0, 0)),
        pl.BlockSpec((1, 128), lambda i: (0, 0)),
        pl.BlockSpec((128, 16), lambda i: (0, 0)),
        pl.BlockSpec((1, 16), lambda i: (0, 0)),
    ],
    out_specs=pl.BlockSpec((B, 16), lambda i: (i, 0)),
    out_shape=jax.ShapeDtypeStruct((N, 16), jnp.float32),
)

def kernel(x, edge_index, W1, b1, W2, b2, Wm1, bm1, Wm2, bm2):
    _app128 = _make_sc_app(True)
    _app64 = _make_sc_app(False)
    _sc_deg = _make_sc_deg()
    src = edge_index[0]
    dst = edge_index[1]
    npad = NW * EPT - E
    srcp = jnp.concatenate([src, jnp.zeros((npad,), jnp.int32)])
    srcp = srcp.reshape(NC, NS, NCHUNK, CH)
    dstp = jnp.concatenate([dst, jnp.full((npad,), TRASH, jnp.int32)])
    dstp = dstp.reshape(NC, NS, NCHUNK, CH)
    srcf = jnp.broadcast_to(
        jnp.concatenate([src, jnp.zeros((npad,), jnp.int32)]).reshape(
            1, NS, NC * NCHUNK, CH), (NC, NS, NC * NCHUNK, CH))
    dstf = jnp.broadcast_to(
        jnp.concatenate([dst, jnp.full((npad,), TRASH, jnp.int32)]).reshape(
            1, NS, NC * NCHUNK, CH), (NC, NS, NC * NCHUNK, CH))

    degp = _sc_deg(dstp)

    xs = _scale0(x, degp)
    p = _app128(xs, srcf, dstf)
    T1, u1 = _comb1(p, degp)
    p = _app128(u1, srcf, dstf)
    T2, u2 = _combk(p, x, degp)
    p = _app128(u2, srcf, dstf)
    T3, u3 = _combk(p, T1, degp)
    p = _app128(u3, srcf, dstf)
    T4, _ = _combk(p, T2, degp)

    w1cat = W1.reshape(5, 128, 256)
    b1r = b1.reshape(1, 256)
    w2cat = jnp.transpose(W2, (1, 0, 2)).reshape(256, 320)
    Y, u4 = _mm1(x, T1, T2, T3, T4, degp, w1cat, b1r, w2cat)

    p = _app64(u4, srcp, dstp)
    c3, u3b = _clen3(p, Y, degp)
    p = _app64(u3b, srcp, dstp)
    c2v, u2b = _clen2(p, Y, Y[4], degp)
    p = _app64(u2b, srcp, dstp)
    c1v, u1b = _clen1(p, Y, c3, degp)
    p = _app64(u1b, srcp, dstp)

    return _final(p, Y[0], c2v, degp, b2.reshape(1, 64),
                  Wm1, bm1.reshape(1, 128), Wm2, bm2.reshape(1, 16))
